# Initial kernel scaffold; baseline (speedup 1.0000x reference)
import jax
import jax.numpy as jnp
from jax.experimental import pallas as pl


def kernel(z, edge_index, edge_length, edge_attr, emblin_W, emblin_b, mlp1_W, mlp1_b, mlp2_W, mlp2_b, lin1_W, lin2_W, lin2_b, lincat_W, lincat_b):
    raise NotImplementedError("write your pallas kernel here")


# speedup vs baseline: 1.0323x; 1.0323x over previous
import math

import jax
import jax.numpy as jnp
from jax.experimental import pallas as pl

_N = 10000
_E = 320000
_D = 128
_G = 100
_INPUT_DIM = 5
_CUTOFF = 10.0
_L = 6


def _ssp(x):
    return jnp.maximum(x, 0.0) + jnp.log1p(jnp.exp(-jnp.abs(x))) - math.log(2.0)


def _we_body(ea_ref, w1_ref, b1_ref, w2_ref, b2_ref, c_ref, out_ref):
    x = jnp.dot(ea_ref[...], w1_ref[...], preferred_element_type=jnp.float32)
    x = _ssp(x + b1_ref[...])
    we = jnp.dot(x, w2_ref[...], preferred_element_type=jnp.float32) + b2_ref[...]
    out_ref[...] = we * c_ref[...]


def _edge_filter(ea, w1, b1, w2, b2, c):
    BE = 1280
    return pl.pallas_call(
        _we_body,
        grid=(_E // BE,),
        in_specs=[
            pl.BlockSpec((BE, _G), lambda i: (i, 0)),
            pl.BlockSpec((_G, _D), lambda i: (0, 0)),
            pl.BlockSpec((1, _D), lambda i: (0, 0)),
            pl.BlockSpec((_D, _D), lambda i: (0, 0)),
            pl.BlockSpec((1, _D), lambda i: (0, 0)),
            pl.BlockSpec((BE, 1), lambda i: (i, 0)),
        ],
        out_specs=pl.BlockSpec((BE, _D), lambda i: (i, 0)),
        out_shape=jax.ShapeDtypeStruct((_E, _D), jnp.float32),
    )(ea, w1, b1.reshape(1, _D), w2, b2.reshape(1, _D), c.reshape(_E, 1))


def kernel(z, edge_index, edge_length, edge_attr, emblin_W, emblin_b,
           mlp1_W, mlp1_b, mlp2_W, mlp2_b, lin1_W, lin2_W, lin2_b,
           lincat_W, lincat_b):
    h = z[:, :_INPUT_DIM] @ emblin_W + emblin_b + z[:, _INPUT_DIM:]
    src = edge_index[0].astype(jnp.int32)
    dst = edge_index[1].astype(jnp.int32)
    C = 0.5 * (jnp.cos(edge_length * jnp.pi / _CUTOFF) + 1.0)
    C = C * (edge_length <= _CUTOFF).astype(jnp.float32)
    C = C * (edge_length >= 0.0).astype(jnp.float32)
    for i in range(_L):
        We = _edge_filter(edge_attr, mlp1_W[i], mlp1_b[i], mlp2_W[i],
                          mlp2_b[i], C)
        xl = h @ lin1_W[i]
        m_ij = jnp.take(xl, src, axis=0) * We
        m_i = jax.ops.segment_sum(m_ij, dst, num_segments=_N)
        xo = _ssp(m_i @ lin2_W[i] + lin2_b[i])
        upd = jnp.concatenate([h, xo], axis=1) @ lincat_W[i] + lincat_b[i]
        h = h + upd
    return h


# speedup vs baseline: 2.1242x; 2.0576x over previous
import functools
import math

import jax
import jax.numpy as jnp
from jax import lax
from jax.experimental import pallas as pl
from jax.experimental.pallas import tpu as pltpu
from jax.experimental.pallas import tpu_sc as plsc

_N = 10000
_E = 320000
_D = 128
_G = 100
_INPUT_DIM = 5
_CUTOFF = 10.0
_L = 6

_NW = 32
_EPW = _E // _NW
_CH = 80
_NCHUNK = _EPW // _CH
_NPAD = 10240
_RPS = _NPAD // 16


def _ssp(x):
    return jnp.maximum(x, 0.0) + jnp.log1p(jnp.exp(-jnp.abs(x))) - math.log(2.0)


def _embed_body(z5_ref, zd_ref, we_ref, be_ref, w1_ref, h_ref, xl_ref):
    h = (jnp.dot(z5_ref[...], we_ref[...], preferred_element_type=jnp.float32)
         + be_ref[...] + zd_ref[...])
    h_ref[...] = h
    xl_ref[...] = jnp.dot(h, w1_ref[...], preferred_element_type=jnp.float32)


def _embed(z5, zd, emblin_W, emblin_b, lin1_W0):
    BN = 1000
    return pl.pallas_call(
        _embed_body,
        grid=(_N // BN,),
        in_specs=[
            pl.BlockSpec((BN, _INPUT_DIM), lambda i: (i, 0)),
            pl.BlockSpec((BN, _D), lambda i: (i, 0)),
            pl.BlockSpec((_INPUT_DIM, _D), lambda i: (0, 0)),
            pl.BlockSpec((1, _D), lambda i: (0, 0)),
            pl.BlockSpec((_D, _D), lambda i: (0, 0)),
        ],
        out_specs=[
            pl.BlockSpec((BN, _D), lambda i: (i, 0)),
            pl.BlockSpec((BN, _D), lambda i: (i, 0)),
        ],
        out_shape=[
            jax.ShapeDtypeStruct((_N, _D), jnp.float32),
            jax.ShapeDtypeStruct((_N, _D), jnp.float32),
        ],
    )(z5, zd, emblin_W, emblin_b.reshape(1, _D), lin1_W0)


def _we_body(ea_ref, w1_ref, b1_ref, w2_ref, b2_ref, el_ref, out_ref):
    x = jnp.dot(ea_ref[...], w1_ref[...], preferred_element_type=jnp.float32)
    x = _ssp(x + b1_ref[...])
    we = jnp.dot(x, w2_ref[...], preferred_element_type=jnp.float32) + b2_ref[...]
    el = el_ref[...]
    c = 0.5 * (jnp.cos(el * (math.pi / _CUTOFF)) + 1.0)
    c = jnp.where((el <= _CUTOFF) & (el >= 0.0), c, 0.0)
    out_ref[...] = we * c


def _edge_filter(ea, w1, b1, w2, b2, el):
    BE = 1280
    return pl.pallas_call(
        _we_body,
        grid=(_E // BE,),
        in_specs=[
            pl.BlockSpec((BE, _G), lambda i: (i, 0)),
            pl.BlockSpec((_G, _D), lambda i: (0, 0)),
            pl.BlockSpec((1, _D), lambda i: (0, 0)),
            pl.BlockSpec((_D, _D), lambda i: (0, 0)),
            pl.BlockSpec((1, _D), lambda i: (0, 0)),
            pl.BlockSpec((BE, 1), lambda i: (i, 0)),
        ],
        out_specs=pl.BlockSpec((BE, _D), lambda i: (i, 0)),
        out_shape=jax.ShapeDtypeStruct((_E, _D), jnp.float32),
    )(ea, w1, b1.reshape(1, _D), w2, b2.reshape(1, _D), el.reshape(_E, 1))


@functools.partial(
    pl.kernel,
    mesh=plsc.VectorSubcoreMesh(core_axis_name="c", subcore_axis_name="s",
                                num_cores=2),
    out_type=jax.ShapeDtypeStruct((2 * _NPAD, _D), jnp.float32),
    scratch_types=[
        pltpu.VMEM((_CH,), jnp.int32),
        pltpu.VMEM((_CH,), jnp.int32),
        pltpu.VMEM((_CH, _D), jnp.float32),
        pltpu.VMEM((_CH, _D), jnp.float32),
        pltpu.VMEM_SHARED((_NPAD, _D), jnp.float32),
        pltpu.SemaphoreType.DMA,
    ],
)
def _cfconv(xl_hbm, we_hbm, src_hbm, dst_hbm, out_hbm,
            src_v, dst_v, rows_v, we_v, acc, sem):
    cid = lax.axis_index("c")
    sid = lax.axis_index("s")
    wid = cid * 16 + sid
    wbase = wid * _EPW
    arow0 = sid * _RPS

    def _zbody(i, _):
        rows_v[i // 8, pl.ds((i % 8) * 16, 16)] = jnp.zeros((16,), jnp.float32)
        return 0
    lax.fori_loop(0, _CH * 8, _zbody, 0)
    for k in range(_RPS // _CH):
        pltpu.sync_copy(rows_v, acc.at[pl.ds(arow0 + k * _CH, _CH), :])
    plsc.subcore_barrier()

    def _chunk(c, _):
        base = wbase + c * _CH
        pltpu.sync_copy(src_hbm.at[pl.ds(base, _CH)], src_v)
        pltpu.sync_copy(we_hbm.at[pl.ds(base, _CH), :], we_v)
        pltpu.async_copy(xl_hbm.at[src_v], rows_v, sem).wait()

        def _mul(r, _):
            for c8 in range(8):
                sl = pl.ds(c8 * 16, 16)
                rows_v[r, sl] = rows_v[r, sl] * we_v[r, sl]
            return 0
        lax.fori_loop(0, _CH, _mul, 0)

        pltpu.sync_copy(dst_hbm.at[pl.ds(base, _CH)], dst_v)
        pltpu.sync_copy(rows_v, acc.at[dst_v], add=True)
        return 0
    lax.fori_loop(0, _NCHUNK, _chunk, 0)

    plsc.subcore_barrier()
    for k in range(_RPS // _CH):
        pltpu.sync_copy(acc.at[pl.ds(arow0 + k * _CH, _CH), :], rows_v)
        pltpu.sync_copy(
            rows_v, out_hbm.at[pl.ds(cid * _NPAD + arow0 + k * _CH, _CH), :])


def _node_body(p_ref, h_ref, w2_ref, b2_ref, wch_ref, wcx_ref, bc_ref,
               w1n_ref, hout_ref, xlout_ref):
    m = p_ref[0] + p_ref[1]
    xo = _ssp(jnp.dot(m, w2_ref[...], preferred_element_type=jnp.float32)
              + b2_ref[...])
    upd = (jnp.dot(h_ref[...], wch_ref[...], preferred_element_type=jnp.float32)
           + jnp.dot(xo, wcx_ref[...], preferred_element_type=jnp.float32)
           + bc_ref[...])
    hn = h_ref[...] + upd
    hout_ref[...] = hn
    xlout_ref[...] = jnp.dot(hn, w1n_ref[...], preferred_element_type=jnp.float32)


def _node_update(p, h, lin2_W, lin2_b, lincat_W, lincat_b, lin1n_W):
    BN = 1000
    p3 = p.reshape(2, _NPAD, _D)
    return pl.pallas_call(
        _node_body,
        grid=(_N // BN,),
        in_specs=[
            pl.BlockSpec((2, BN, _D), lambda i: (0, i, 0)),
            pl.BlockSpec((BN, _D), lambda i: (i, 0)),
            pl.BlockSpec((_D, _D), lambda i: (0, 0)),
            pl.BlockSpec((1, _D), lambda i: (0, 0)),
            pl.BlockSpec((_D, _D), lambda i: (0, 0)),
            pl.BlockSpec((_D, _D), lambda i: (0, 0)),
            pl.BlockSpec((1, _D), lambda i: (0, 0)),
            pl.BlockSpec((_D, _D), lambda i: (0, 0)),
        ],
        out_specs=[
            pl.BlockSpec((BN, _D), lambda i: (i, 0)),
            pl.BlockSpec((BN, _D), lambda i: (i, 0)),
        ],
        out_shape=[
            jax.ShapeDtypeStruct((_N, _D), jnp.float32),
            jax.ShapeDtypeStruct((_N, _D), jnp.float32),
        ],
    )(p3, h, lin2_W, lin2_b.reshape(1, _D), lincat_W[:_D], lincat_W[_D:],
      lincat_b.reshape(1, _D), lin1n_W)


def kernel(z, edge_index, edge_length, edge_attr, emblin_W, emblin_b,
           mlp1_W, mlp1_b, mlp2_W, mlp2_b, lin1_W, lin2_W, lin2_b,
           lincat_W, lincat_b):
    src = edge_index[0].astype(jnp.int32)
    dst = edge_index[1].astype(jnp.int32)
    h, xl = _embed(z[:, :_INPUT_DIM], z[:, _INPUT_DIM:], emblin_W, emblin_b,
                   lin1_W[0])
    for i in range(_L):
        We = _edge_filter(edge_attr, mlp1_W[i], mlp1_b[i], mlp2_W[i],
                          mlp2_b[i], edge_length)
        p = _cfconv(xl, We, src, dst)
        w1n = lin1_W[(i + 1) % _L]
        h, xl = _node_update(p, h, lin2_W[i], lin2_b[i], lincat_W[i],
                             lincat_b[i], w1n)
    return h
